# baseline (speedup 1.0000x reference)
import functools

import jax
import jax.numpy as jnp
from jax.experimental import pallas as pl
from jax.experimental.pallas import tpu as pltpu

_K = 8
_BIG = 1e30


def _block_kernel(xp_ref, xpT_ref, xt_ref, xtT_ref, out_ref, *,
                  n_pts, blk_rows):
    i = pl.program_id(1)

    @pl.when(i == 0)
    def _init():
        out_ref[...] = jnp.zeros_like(out_ref)

    row0 = i * blk_rows
    col_ids = jax.lax.broadcasted_iota(jnp.int32, (blk_rows, n_pts), 1)
    row_ids = row0 + jax.lax.broadcasted_iota(jnp.int32, (blk_rows, n_pts), 0)

    def select_neighbors(x_ref, xT_ref):
        xaT = xT_ref[0, :, :]
        xb = x_ref[0, pl.ds(row0, blk_rows), :]
        xax = xaT[0:1, :]
        xay = xaT[1:2, :]
        xaz = xaT[2:3, :]
        sqb = jnp.sum(xb * xb, axis=1, keepdims=True)
        sqa = jnp.sum(xaT * xaT, axis=0, keepdims=True)
        cross = jnp.dot(xb, xaT, preferred_element_type=jnp.float32)
        d2 = sqb + sqa - 2.0 * cross
        dist = jnp.sqrt(jnp.maximum(d2, 1e-12))
        dist = jnp.where(col_ids == row_ids, _BIG, dist)
        dists, nbrs = [], []
        for _ in range(_K):
            m = jnp.min(dist, axis=1, keepdims=True)
            hit = dist == m
            nx = jnp.min(jnp.where(hit, xax, _BIG), axis=1, keepdims=True)
            ny = jnp.min(jnp.where(hit, xay, _BIG), axis=1, keepdims=True)
            nz = jnp.min(jnp.where(hit, xaz, _BIG), axis=1, keepdims=True)
            dists.append(m)
            nbrs.append(jnp.concatenate([nx, ny, nz], axis=1))
            dist = jnp.where(hit, _BIG, dist)
        return xb, dists, nbrs

    xpb, pdists, pnbrs = select_neighbors(xp_ref, xpT_ref)
    xtb, tdists, tnbrs = select_neighbors(xt_ref, xtT_ref)

    comps = [(0, 0), (1, 1), (2, 2), (0, 1), (0, 2), (1, 2)]
    pcov = [jnp.zeros((blk_rows, 1), jnp.float32) for _ in comps]
    tcov = [jnp.zeros((blk_rows, 1), jnp.float32) for _ in comps]
    sim_acc = jnp.zeros((blk_rows, 1), jnp.float32)
    for k in range(_K):
        pv = pnbrs[k] - xpb
        tv = tnbrs[k] - xtb
        pn = jnp.maximum(jnp.sqrt(jnp.sum(pv * pv, axis=1, keepdims=True)),
                         1e-12)
        tn = jnp.maximum(jnp.sqrt(jnp.sum(tv * tv, axis=1, keepdims=True)),
                         1e-12)
        sim_acc += jnp.sum((pv / pn) * (tv / tn), axis=1, keepdims=True)
        for c, (a, bb) in enumerate(comps):
            pcov[c] = pcov[c] + pv[:, a:a + 1] * pv[:, bb:bb + 1]
            tcov[c] = tcov[c] + tv[:, a:a + 1] * tv[:, bb:bb + 1]

    inv_k = jnp.float32(1.0 / _K)
    pdens = sum(pdists) * inv_k
    tdens = sum(tdists) * inv_k
    ddiff = pdens - tdens

    dc = [pcov[c] - tcov[c] for c in range(6)]
    fro2 = (dc[0] * dc[0] + dc[1] * dc[1] + dc[2] * dc[2]
            + 2.0 * (dc[3] * dc[3] + dc[4] * dc[4] + dc[5] * dc[5]))
    cov_row = jnp.sqrt(fro2) * inv_k

    dens_sum = jnp.sum(ddiff * ddiff, axis=0, keepdims=True)
    sim_sum = jnp.sum(sim_acc, axis=0, keepdims=True)
    cov_sum = jnp.sum(cov_row, axis=0, keepdims=True)

    lane = jax.lax.broadcasted_iota(jnp.int32, (1, 128), 1)
    packed = (jnp.where(lane == 0, dens_sum, 0.0)
              + jnp.where(lane == 1, sim_sum, 0.0)
              + jnp.where(lane == 2, cov_sum, 0.0))
    out_ref[0, :, :] += packed


@jax.jit
def kernel(pred, target):
    B, N, _ = pred.shape
    R = 256
    nb = N // R
    predT = jnp.swapaxes(pred, 1, 2)
    targetT = jnp.swapaxes(target, 1, 2)

    full = pl.BlockSpec((1, N, 3), lambda b, i: (b, 0, 0))
    fullT = pl.BlockSpec((1, 3, N), lambda b, i: (b, 0, 0))
    outspec = pl.BlockSpec((1, 1, 128), lambda b, i: (b, 0, 0))

    out = pl.pallas_call(
        functools.partial(_block_kernel, n_pts=N, blk_rows=R),
        grid=(B, nb),
        in_specs=[full, fullT, full, fullT],
        out_specs=outspec,
        out_shape=jax.ShapeDtypeStruct((B, 1, 128), jnp.float32),
        compiler_params=pltpu.CompilerParams(
            dimension_semantics=("parallel", "arbitrary")),
    )(pred, predT, target, targetT)

    sums = jnp.sum(out[:, 0, :], axis=0)
    dens_t = sums[0]
    sim_t = sums[1]
    cov_t = sums[2]
    alpha = jnp.float32(0.5)
    loss = (dens_t / N
            + alpha * (B - sim_t / (N * _K))
            + (1.0 - alpha) * cov_t / N) / B
    return loss


# speedup vs baseline: 1.4247x; 1.4247x over previous
import functools

import jax
import jax.numpy as jnp
from jax.experimental import pallas as pl
from jax.experimental.pallas import tpu as pltpu

_K = 8
_BIG = 1e30


def _block_kernel(xp_ref, xpT_ref, xt_ref, xtT_ref, out_ref, *,
                  n_pts, blk_rows):
    i = pl.program_id(1)

    @pl.when(i == 0)
    def _init():
        out_ref[...] = jnp.zeros_like(out_ref)

    row0 = i * blk_rows
    col_ids = jax.lax.broadcasted_iota(jnp.int32, (blk_rows, n_pts), 1)
    row_ids = row0 + jax.lax.broadcasted_iota(jnp.int32, (blk_rows, n_pts), 0)

    def select_neighbors(x_ref, xT_ref):
        xa4 = x_ref[0, :, :]
        xaT = xT_ref[0, :, :]
        xb = x_ref[0, pl.ds(row0, blk_rows), 0:3]
        sqb = jnp.sum(xb * xb, axis=1, keepdims=True)
        sqa = jnp.sum(xaT * xaT, axis=0, keepdims=True)
        cross = jnp.dot(xb, xaT, preferred_element_type=jnp.float32)
        d2 = sqb + sqa - 2.0 * cross
        d2 = jnp.where(col_ids == row_ids, _BIG, d2)
        dists, nbrs = [], []
        for _ in range(_K):
            m2 = jnp.min(d2, axis=1, keepdims=True)
            hf = jnp.where(d2 == m2, jnp.float32(1.0), jnp.float32(0.0))
            g = jnp.dot(hf, xa4, preferred_element_type=jnp.float32)
            d2 = d2 + hf * _BIG
            dists.append(jnp.sqrt(jnp.maximum(m2, 1e-12)))
            nbrs.append(g[:, 0:3] * (1.0 / g[:, 3:4]))
        return xb, dists, nbrs

    xpb, pdists, pnbrs = select_neighbors(xp_ref, xpT_ref)
    xtb, tdists, tnbrs = select_neighbors(xt_ref, xtT_ref)

    comps = [(0, 0), (1, 1), (2, 2), (0, 1), (0, 2), (1, 2)]
    pcov = [jnp.zeros((blk_rows, 1), jnp.float32) for _ in comps]
    tcov = [jnp.zeros((blk_rows, 1), jnp.float32) for _ in comps]
    sim_acc = jnp.zeros((blk_rows, 1), jnp.float32)
    for k in range(_K):
        pv = pnbrs[k] - xpb
        tv = tnbrs[k] - xtb
        pn = jnp.maximum(jnp.sqrt(jnp.sum(pv * pv, axis=1, keepdims=True)),
                         1e-12)
        tn = jnp.maximum(jnp.sqrt(jnp.sum(tv * tv, axis=1, keepdims=True)),
                         1e-12)
        sim_acc += jnp.sum((pv / pn) * (tv / tn), axis=1, keepdims=True)
        for c, (a, bb) in enumerate(comps):
            pcov[c] = pcov[c] + pv[:, a:a + 1] * pv[:, bb:bb + 1]
            tcov[c] = tcov[c] + tv[:, a:a + 1] * tv[:, bb:bb + 1]

    inv_k = jnp.float32(1.0 / _K)
    pdens = sum(pdists) * inv_k
    tdens = sum(tdists) * inv_k
    ddiff = pdens - tdens

    dc = [pcov[c] - tcov[c] for c in range(6)]
    fro2 = (dc[0] * dc[0] + dc[1] * dc[1] + dc[2] * dc[2]
            + 2.0 * (dc[3] * dc[3] + dc[4] * dc[4] + dc[5] * dc[5]))
    cov_row = jnp.sqrt(fro2) * inv_k

    dens_sum = jnp.sum(ddiff * ddiff, axis=0, keepdims=True)
    sim_sum = jnp.sum(sim_acc, axis=0, keepdims=True)
    cov_sum = jnp.sum(cov_row, axis=0, keepdims=True)

    lane = jax.lax.broadcasted_iota(jnp.int32, (1, 128), 1)
    packed = (jnp.where(lane == 0, dens_sum, 0.0)
              + jnp.where(lane == 1, sim_sum, 0.0)
              + jnp.where(lane == 2, cov_sum, 0.0))
    out_ref[0, :, :] += packed


@jax.jit
def kernel(pred, target):
    B, N, _ = pred.shape
    R = 256
    nb = N // R
    ones = jnp.ones((B, N, 1), jnp.float32)
    pred4 = jnp.concatenate([pred, ones], axis=2)
    target4 = jnp.concatenate([target, ones], axis=2)
    predT = jnp.swapaxes(pred, 1, 2)
    targetT = jnp.swapaxes(target, 1, 2)

    full = pl.BlockSpec((1, N, 4), lambda b, i: (b, 0, 0))
    fullT = pl.BlockSpec((1, 3, N), lambda b, i: (b, 0, 0))
    outspec = pl.BlockSpec((1, 1, 128), lambda b, i: (b, 0, 0))

    out = pl.pallas_call(
        functools.partial(_block_kernel, n_pts=N, blk_rows=R),
        grid=(B, nb),
        in_specs=[full, fullT, full, fullT],
        out_specs=outspec,
        out_shape=jax.ShapeDtypeStruct((B, 1, 128), jnp.float32),
        compiler_params=pltpu.CompilerParams(
            dimension_semantics=("parallel", "arbitrary")),
    )(pred4, predT, target4, targetT)

    sums = jnp.sum(out[:, 0, :], axis=0)
    dens_t = sums[0]
    sim_t = sums[1]
    cov_t = sums[2]
    alpha = jnp.float32(0.5)
    loss = (dens_t / N
            + alpha * (B - sim_t / (N * _K))
            + (1.0 - alpha) * cov_t / N) / B
    return loss
